# baseline (speedup 1.0000x reference)
import functools

import jax
import jax.numpy as jnp
from jax import lax
from jax.experimental import pallas as pl
from jax.experimental.pallas import tpu as pltpu
from jax.experimental.pallas import tpu_sc as plsc

B = 16384
NUM_ITEMS = 1000000
_SPLIT = 999424
_TAIL = 1024

_info = plsc.get_sparse_core_info()
_NC, _NS, _NL = _info.num_cores, _info.num_subcores, _info.num_lanes
_NW = _NC * _NS
_CHUNK = 128
_PER_W = B // _NW
_NCH = _PER_W // _CHUNK


@functools.partial(
    pl.kernel,
    mesh=plsc.VectorSubcoreMesh(core_axis_name="c", subcore_axis_name="s"),
    out_type=jax.ShapeDtypeStruct((B,), jnp.float32),
    scratch_types=[
        pltpu.VMEM((_NCH, _CHUNK), jnp.int32),
        pltpu.VMEM((_NCH, _CHUNK), jnp.int32),
        pltpu.VMEM((_NCH, _CHUNK), jnp.int32),
        pltpu.VMEM((_PER_W,), jnp.float32),
        pltpu.VMEM((_PER_W,), jnp.float32),
        pltpu.VMEM((_PER_W,), jnp.float32),
        pltpu.SemaphoreType.DMA,
        pltpu.SemaphoreType.DMA,
    ],
)
def _bias_gather(head_hbm, tail_hbm, idx_hbm, out_hbm,
                 idx_v, hidx_v, tidx_v, hrow_v, trow_v, out_v, isem, gsem):
    wid = lax.axis_index("s") * _NC + lax.axis_index("c")
    idx_copies = [
        pltpu.async_copy(idx_hbm.at[wid, j], idx_v.at[j], isem)
        for j in range(_NCH)
    ]
    gathers = []
    for j in range(_NCH):
        idx_copies[j].wait()
        for k in range(_CHUNK // _NL):
            sl = pl.ds(k * _NL, _NL)
            v = idx_v[j, sl]
            hidx_v[j, sl] = lax.min(v, _SPLIT - 1)
            tidx_v[j, sl] = lax.max(v - _SPLIT, 0)
        gathers.append(
            pltpu.async_copy(
                head_hbm.at[hidx_v.at[j]],
                hrow_v.at[pl.ds(j * _CHUNK, _CHUNK)],
                gsem,
            )
        )
        gathers.append(
            pltpu.async_copy(
                tail_hbm.at[tidx_v.at[j]],
                trow_v.at[pl.ds(j * _CHUNK, _CHUNK)],
                gsem,
            )
        )
    for g in gathers:
        g.wait()
    for j in range(_NCH):
        for k in range(_CHUNK // _NL):
            sl = pl.ds(j * _CHUNK + k * _NL, _NL)
            v = idx_v[j, pl.ds(k * _NL, _NL)]
            out_v[sl] = jnp.where(v < _SPLIT, hrow_v[sl], trow_v[sl])
    pltpu.sync_copy(out_v, out_hbm.at[pl.ds(wid * _PER_W, _PER_W)])


def kernel(item_sequences, item_ids, item_biases):
    idx = item_ids.reshape(_NW, _NCH, _CHUNK)
    head = lax.slice(item_biases, (0, 0), (_SPLIT, 1)).reshape(_SPLIT)
    tail = lax.slice(item_biases, (_SPLIT, 0), (NUM_ITEMS, 1))
    tail = jnp.pad(tail.reshape(NUM_ITEMS - _SPLIT), (0, _TAIL - (NUM_ITEMS - _SPLIT)))
    out = _bias_gather(head, tail, idx)
    return out.reshape(B, 1)


# speedup vs baseline: 4.5982x; 4.5982x over previous
import functools

import jax
import jax.numpy as jnp
from jax import lax
from jax.experimental import pallas as pl
from jax.experimental.pallas import tpu as pltpu
from jax.experimental.pallas import tpu_sc as plsc

B = 16384
NUM_ITEMS = 1000000
_SPLIT = 999424
_PADDED = 1000448

_info = plsc.get_sparse_core_info()
_NC, _NS = _info.num_cores, _info.num_subcores
_NW = _NC * _NS
_CHUNK = 128
_PER_W = B // _NW
_NCH = _PER_W // _CHUNK


@functools.partial(
    pl.kernel,
    mesh=plsc.VectorSubcoreMesh(core_axis_name="c", subcore_axis_name="s"),
    out_type=jax.ShapeDtypeStruct((B,), jnp.float32),
    scratch_types=[
        pltpu.VMEM((_NCH, _CHUNK), jnp.int32),
        pltpu.VMEM((_PER_W,), jnp.float32),
        pltpu.SemaphoreType.DMA,
        pltpu.SemaphoreType.DMA,
    ],
    compiler_params=pltpu.CompilerParams(
        skip_device_barrier=True, disable_bounds_checks=True
    ),
)
def _bias_gather(table_hbm, idx_hbm, out_hbm, idx_v, rows_v, isem, gsem):
    wid = lax.axis_index("s") * _NC + lax.axis_index("c")
    idx_copies = [
        pltpu.async_copy(idx_hbm.at[wid, j], idx_v.at[j], isem)
        for j in range(_NCH)
    ]
    gathers = []
    for j in range(_NCH):
        idx_copies[j].wait()
        gathers.append(
            pltpu.async_copy(
                table_hbm.at[idx_v.at[j]],
                rows_v.at[pl.ds(j * _CHUNK, _CHUNK)],
                gsem,
            )
        )
    for g in gathers:
        g.wait()
    pltpu.sync_copy(rows_v, out_hbm.at[pl.ds(wid * _PER_W, _PER_W)])


def kernel(item_sequences, item_ids, item_biases):
    idx = item_ids.reshape(_NW, _NCH, _CHUNK)
    head = lax.slice(item_biases, (0, 0), (_SPLIT, 1)).reshape(_SPLIT)
    tail = lax.slice(item_biases, (_SPLIT, 0), (NUM_ITEMS, 1)).reshape(NUM_ITEMS - _SPLIT)
    tail = jnp.pad(tail, (0, _PADDED - NUM_ITEMS))
    table = jnp.concatenate([head, tail], axis=0)
    out = _bias_gather(table, idx)
    return out.reshape(B, 1)


# speedup vs baseline: 4.5997x; 1.0003x over previous
import functools

import jax
import jax.numpy as jnp
from jax import lax
from jax.experimental import pallas as pl
from jax.experimental.pallas import tpu as pltpu
from jax.experimental.pallas import tpu_sc as plsc

B = 16384
NUM_ITEMS = 1000000
_SPLIT = 999424
_PADDED = 1000448

_info = plsc.get_sparse_core_info()
_NC, _NS = _info.num_cores, _info.num_subcores
_NW = _NC * _NS
_CHUNK = 128
_PER_W = B // _NW
_NCH = _PER_W // _CHUNK


@functools.partial(
    pl.kernel,
    mesh=plsc.VectorSubcoreMesh(core_axis_name="c", subcore_axis_name="s"),
    out_type=jax.ShapeDtypeStruct((B,), jnp.float32),
    scratch_types=[
        pltpu.VMEM((_NCH, _CHUNK), jnp.int32),
        pltpu.VMEM((_PER_W,), jnp.float32),
        pltpu.SemaphoreType.DMA,
        pltpu.SemaphoreType.DMA,
    ],
)
def _bias_gather(table_hbm, idx_hbm, out_hbm, idx_v, rows_v, isem, gsem):
    wid = lax.axis_index("s") * _NC + lax.axis_index("c")
    idx_copies = [
        pltpu.async_copy(idx_hbm.at[wid, j], idx_v.at[j], isem)
        for j in range(_NCH)
    ]
    gathers = []
    for j in range(_NCH):
        idx_copies[j].wait()
        gathers.append(
            pltpu.async_copy(
                table_hbm.at[idx_v.at[j]],
                rows_v.at[pl.ds(j * _CHUNK, _CHUNK)],
                gsem,
            )
        )
    for g in gathers:
        g.wait()
    pltpu.sync_copy(rows_v, out_hbm.at[pl.ds(wid * _PER_W, _PER_W)])


def kernel(item_sequences, item_ids, item_biases):
    idx = item_ids.reshape(_NW, _NCH, _CHUNK)
    head = lax.slice(item_biases, (0, 0), (_SPLIT, 1)).reshape(_SPLIT)
    tail = lax.slice(item_biases, (_SPLIT, 0), (NUM_ITEMS, 1)).reshape(NUM_ITEMS - _SPLIT)
    tail = jnp.pad(tail, (0, _PADDED - NUM_ITEMS))
    table = jnp.concatenate([head, tail], axis=0)
    out = _bias_gather(table, idx)
    return out.reshape(B, 1)
